# baseline (speedup 1.0000x reference)
import functools

import jax
import jax.numpy as jnp
from jax import lax
from jax.experimental import pallas as pl
from jax.experimental.pallas import tpu as pltpu
from jax.experimental.pallas import tpu_sc as plsc

B = 64
S = 2048
E = 128

NC = 2
NS = 16
NW = NC * NS

PG = 8
BG = NW // PG
BATCH_PER_G = B // BG
POS_PER_P = S // PG
CHUNK = POS_PER_P
KSUB = CHUNK // 128


def _tpe_body(x_hbm, tok_hbm, pos_hbm, lin_hbm, out_hbm,
              idx_v, rows_v, pos_v, lin_v):
    wid = lax.axis_index("s") * NC + lax.axis_index("c")
    g = wid // PG
    p = wid % PG
    pos_base = p * POS_PER_P

    pltpu.sync_copy(pos_hbm.at[pl.ds(pos_base, POS_PER_P)], pos_v)
    pltpu.sync_copy(lin_hbm, lin_v)

    @pl.loop(0, BATCH_PER_G)
    def _(b):
        batch = g * BATCH_PER_G + b
        pltpu.sync_copy(x_hbm.at[batch].at[pl.ds(p * KSUB, KSUB)], idx_v)
        pltpu.sync_copy(pos_hbm.at[pl.ds(pos_base, CHUNK)], rows_v)
        for j in range(KSUB):
            pltpu.sync_copy(tok_hbm.at[idx_v.at[j]],
                            rows_v.at[pl.ds(j * 128, 128)], add=True)
        row_off = batch * S + pos_base
        pltpu.sync_copy(rows_v, out_hbm.at[pl.ds(row_off, CHUNK)])


def kernel(x, token_table, pos_table):
    xi = x.reshape(B, S // 128, 128).astype(jnp.int32)
    lin = jnp.arange(CHUNK, dtype=jnp.int32).reshape(KSUB, 128)
    mesh = plsc.VectorSubcoreMesh(core_axis_name="c", subcore_axis_name="s")
    f = pl.kernel(
        _tpe_body,
        out_type=jax.ShapeDtypeStruct((B * S, E), jnp.float32),
        mesh=mesh,
        scratch_types=[
            pltpu.VMEM((KSUB, 128), jnp.int32),
            pltpu.VMEM((CHUNK, E), jnp.float32),
            pltpu.VMEM((POS_PER_P, E), jnp.float32),
            pltpu.VMEM((KSUB, 128), jnp.int32),
        ],
    )
    out = f(xi, token_table, pos_table, lin)
    return out.reshape(B, S, E)


# speedup vs baseline: 1.1411x; 1.1411x over previous
import functools

import jax
import jax.numpy as jnp
from jax import lax
from jax.experimental import pallas as pl
from jax.experimental.pallas import tpu as pltpu
from jax.experimental.pallas import tpu_sc as plsc

B = 64
S = 2048
E = 128

NC = 2
NS = 16
NW = NC * NS

PG = 8
BG = NW // PG
BATCH_PER_G = B // BG
POS_PER_P = S // PG
CHUNK = POS_PER_P
KSUB = CHUNK // 128


def _tpe_body(x_hbm, tok_hbm, pos_hbm, out_hbm,
              idx_v, rows_v, pos_sh):
    sid = lax.axis_index("s")
    wid = sid * NC + lax.axis_index("c")
    g = wid // PG
    p = wid % PG
    pos_base = p * POS_PER_P

    pltpu.sync_copy(pos_hbm.at[pl.ds(sid * 128, 128)],
                    pos_sh.at[pl.ds(sid * 128, 128)])
    plsc.subcore_barrier()

    @pl.loop(0, BATCH_PER_G)
    def _(b):
        batch = g * BATCH_PER_G + b
        pltpu.sync_copy(x_hbm.at[batch].at[pl.ds(p * KSUB, KSUB)], idx_v)
        pltpu.sync_copy(pos_sh.at[pl.ds(pos_base, CHUNK)], rows_v)
        for j in range(KSUB):
            pltpu.sync_copy(tok_hbm.at[idx_v.at[j]],
                            rows_v.at[pl.ds(j * 128, 128)], add=True)
        row_off = batch * S + pos_base
        pltpu.sync_copy(rows_v, out_hbm.at[pl.ds(row_off, CHUNK)])


def kernel(x, token_table, pos_table):
    xi = x.reshape(B, S // 128, 128).astype(jnp.int32)
    mesh = plsc.VectorSubcoreMesh(core_axis_name="c", subcore_axis_name="s")
    f = pl.kernel(
        _tpe_body,
        out_type=jax.ShapeDtypeStruct((B * S, E), jnp.float32),
        mesh=mesh,
        scratch_types=[
            pltpu.VMEM((KSUB, 128), jnp.int32),
            pltpu.VMEM((CHUNK, E), jnp.float32),
            pltpu.VMEM_SHARED((S, E), jnp.float32),
        ],
    )
    out = f(xi, token_table, pos_table)
    return out.reshape(B, S, E)


# speedup vs baseline: 1.3014x; 1.1405x over previous
import functools

import jax
import jax.numpy as jnp
from jax import lax
from jax.experimental import pallas as pl
from jax.experimental.pallas import tpu as pltpu
from jax.experimental.pallas import tpu_sc as plsc

B = 64
S = 2048
E = 128

NC = 2
NS = 16
NW = NC * NS

PG = 8
BG = NW // PG
BATCH_PER_G = B // BG
POS_PER_P = S // PG
CHUNK = POS_PER_P
KSUB = CHUNK // 128


def _tpe_body(x_hbm, tok_hbm, pos_hbm, out_hbm,
              idx_v, rows_v, pos_sh, sem0, sem1):
    sid = lax.axis_index("s")
    wid = sid * NC + lax.axis_index("c")
    g = wid // PG
    p = wid % PG
    pos_base = p * POS_PER_P
    sems = (sem0, sem1)

    pltpu.sync_copy(pos_hbm.at[pl.ds(sid * 128, 128)],
                    pos_sh.at[pl.ds(sid * 128, 128)])
    plsc.subcore_barrier()

    for b in range(BATCH_PER_G):
        r = b % 2
        batch = g * BATCH_PER_G + b
        row_off = batch * S + pos_base
        buf = rows_v.at[r]
        out_slc = out_hbm.at[pl.ds(row_off, CHUNK)]

        if b >= 2:
            pltpu.make_async_copy(buf, out_slc, sems[r]).wait()

        pltpu.sync_copy(x_hbm.at[batch].at[pl.ds(p * KSUB, KSUB)], idx_v)
        pltpu.sync_copy(pos_sh.at[pl.ds(pos_base, CHUNK)], buf)
        for j in range(KSUB):
            pltpu.sync_copy(tok_hbm.at[idx_v.at[j]],
                            buf.at[pl.ds(j * 128, 128)], add=True)
        pltpu.async_copy(buf, out_slc, sems[r])

    for b in range(BATCH_PER_G - 2, BATCH_PER_G):
        r = b % 2
        batch = g * BATCH_PER_G + b
        row_off = batch * S + pos_base
        pltpu.make_async_copy(rows_v.at[r],
                              out_hbm.at[pl.ds(row_off, CHUNK)],
                              sems[r]).wait()


def kernel(x, token_table, pos_table):
    xi = x.reshape(B, S // 128, 128).astype(jnp.int32)
    mesh = plsc.VectorSubcoreMesh(core_axis_name="c", subcore_axis_name="s")
    f = pl.kernel(
        _tpe_body,
        out_type=jax.ShapeDtypeStruct((B * S, E), jnp.float32),
        mesh=mesh,
        scratch_types=[
            pltpu.VMEM((KSUB, 128), jnp.int32),
            pltpu.VMEM((2, CHUNK, E), jnp.float32),
            pltpu.VMEM_SHARED((S, E), jnp.float32),
            pltpu.SemaphoreType.DMA,
            pltpu.SemaphoreType.DMA,
        ],
    )
    out = f(xi, token_table, pos_table)
    return out.reshape(B, S, E)


# speedup vs baseline: 1.9240x; 1.4784x over previous
import jax
import jax.numpy as jnp
from jax import lax
from jax.experimental import pallas as pl
from jax.experimental.pallas import tpu as pltpu
from jax.experimental.pallas import tpu_sc as plsc

B = 64
S = 2048
E = 128

NC = 2
NS = 16
NW = NC * NS

PG = 8
BG = NW // PG
BATCH_PER_G = B // BG
POS_PER_P = S // PG
CHUNK = POS_PER_P
KSUB = CHUNK // 128
NBUF = 3


def _tpe_body(xt_hbm, tok_hbm, pos_hbm, out_hbm,
              idx_all, rows_v, pos_sh,
              g0, g1, g2, s0, s1, s2):
    sid = lax.axis_index("s")
    wid = sid * NC + lax.axis_index("c")
    g = wid // PG
    p = wid % PG
    pos_base = p * POS_PER_P
    gat_sems = (g0, g1, g2)
    st_sems = (s0, s1, s2)

    pltpu.sync_copy(pos_hbm.at[pl.ds(sid * 128, 128)],
                    pos_sh.at[pl.ds(sid * 128, 128)])

    pltpu.sync_copy(xt_hbm.at[wid], idx_all)
    plsc.subcore_barrier()

    def out_slc(c):
        batch = g * BATCH_PER_G + c
        return out_hbm.at[pl.ds(batch * S + pos_base, CHUNK)]

    def stage_a(c):
        r = c % NBUF
        buf = rows_v.at[r]
        if c >= NBUF:
            pltpu.make_async_copy(rows_v.at[r], out_slc(c - NBUF),
                                  st_sems[r]).wait()
        pltpu.sync_copy(pos_sh.at[pl.ds(pos_base, CHUNK)], buf)
        for j in range(KSUB):
            pltpu.async_copy(tok_hbm.at[idx_all.at[c * KSUB + j]],
                             buf.at[pl.ds(j * 128, 128)], gat_sems[r],
                             add=True)

    def stage_b(c):
        r = c % NBUF
        buf = rows_v.at[r]
        for j in range(KSUB):
            pltpu.make_async_copy(tok_hbm.at[idx_all.at[c * KSUB + j]],
                                  buf.at[pl.ds(j * 128, 128)],
                                  gat_sems[r]).wait()
        pltpu.async_copy(buf, out_slc(c), st_sems[r])

    stage_a(0)
    for c in range(BATCH_PER_G):
        if c + 1 < BATCH_PER_G:
            stage_a(c + 1)
        stage_b(c)

    for c in range(BATCH_PER_G - NBUF, BATCH_PER_G):
        r = c % NBUF
        pltpu.make_async_copy(rows_v.at[r], out_slc(c), st_sems[r]).wait()


def kernel(x, token_table, pos_table):
    xi = x.astype(jnp.int32).reshape(BG, BATCH_PER_G, PG, KSUB, 128)
    xt = xi.transpose(0, 2, 1, 3, 4).reshape(NW, BATCH_PER_G * KSUB, 128)
    mesh = plsc.VectorSubcoreMesh(core_axis_name="c", subcore_axis_name="s")
    f = pl.kernel(
        _tpe_body,
        out_type=jax.ShapeDtypeStruct((B * S, E), jnp.float32),
        mesh=mesh,
        scratch_types=[
            pltpu.VMEM((BATCH_PER_G * KSUB, 128), jnp.int32),
            pltpu.VMEM((NBUF, CHUNK, E), jnp.float32),
            pltpu.VMEM_SHARED((S, E), jnp.float32),
            pltpu.SemaphoreType.DMA,
            pltpu.SemaphoreType.DMA,
            pltpu.SemaphoreType.DMA,
            pltpu.SemaphoreType.DMA,
            pltpu.SemaphoreType.DMA,
            pltpu.SemaphoreType.DMA,
        ],
    )
    out = f(xt, token_table, pos_table)
    return out.reshape(B, S, E)
